# Initial kernel scaffold; baseline (speedup 1.0000x reference)
import jax
import jax.numpy as jnp
from jax.experimental import pallas as pl


def kernel(gt_boxes, classes, cls_logits_0, cnt_logits_0, reg_preds_0, cls_logits_1, cnt_logits_1, reg_preds_1, cls_logits_2, cnt_logits_2, reg_preds_2, cls_logits_3, cnt_logits_3, reg_preds_3, cls_logits_4, cnt_logits_4, reg_preds_4):
    raise NotImplementedError("write your pallas kernel here")


# speedup vs baseline: 3.4368x; 3.4368x over previous
import functools

import jax
import jax.numpy as jnp
from jax import lax
from jax.experimental import pallas as pl
from jax.experimental.pallas import tpu as pltpu
from jax.experimental.pallas import tpu_sc as plsc

STRIDES = (8, 16, 32, 64, 128)
LIMITS = ((-1.0, 64.0), (64.0, 128.0), (128.0, 256.0), (256.0, 512.0),
          (512.0, 999999.0))
IMG = 1024
B = 8
M = 50
RADIU_RATIO = 1.5
BIG = 999999999.0

TOT = sum((IMG // s) ** 2 for s in STRIDES)
NW = 32
CHUNK = 5504
NPAD = 4 * CHUNK
NVEC = CHUNK // 16
NROW = 7
ROWLEN = M * 16
BOXSZ = NROW * ROWLEN


def _point_data():
    xs, ys, los, his, rads = [], [], [], [], []
    for lvl, s in enumerate(STRIDES):
        hw = IMG // s
        shifts = jnp.arange(0, hw * s, s, dtype=jnp.float32) + s // 2
        sy, sx = jnp.meshgrid(shifts, shifts, indexing='ij')
        xs.append(sx.reshape(-1))
        ys.append(sy.reshape(-1))
        los.append(jnp.full((hw * hw,), LIMITS[lvl][0], jnp.float32))
        his.append(jnp.full((hw * hw,), LIMITS[lvl][1], jnp.float32))
        rads.append(jnp.full((hw * hw,), s * RADIU_RATIO, jnp.float32))
    pad = NPAD - TOT
    z = jnp.zeros((pad,), jnp.float32)
    return (jnp.concatenate(xs + [z]), jnp.concatenate(ys + [z]),
            jnp.concatenate(los + [z]), jnp.concatenate(his + [z]),
            jnp.concatenate(rads + [z]))


def _tile_body(px_h, py_h, lo_h, hi_h, rad_h, box_h,
               cls_o, cnt_o, l_o, t_o, r_o, b_o,
               pxv, pyv, lov, hiv, radv, boxv,
               clsv, cntv, lv, tv, rv, bv):
    wid = lax.axis_index("s") * 2 + lax.axis_index("c")
    poff = (wid % 4) * CHUNK
    ooff = wid * CHUNK
    boff = (wid // 4) * BOXSZ

    pltpu.sync_copy(px_h.at[pl.ds(poff, CHUNK)], pxv)
    pltpu.sync_copy(py_h.at[pl.ds(poff, CHUNK)], pyv)
    pltpu.sync_copy(lo_h.at[pl.ds(poff, CHUNK)], lov)
    pltpu.sync_copy(hi_h.at[pl.ds(poff, CHUNK)], hiv)
    pltpu.sync_copy(rad_h.at[pl.ds(poff, CHUNK)], radv)
    pltpu.sync_copy(box_h.at[pl.ds(boff, BOXSZ)], boxv)

    lane = lax.broadcasted_iota(jnp.int32, (16,), 0)
    big = jnp.full((16,), BIG, jnp.float32)

    def body(i, _):
        px = pxv[pl.ds(i * 16, 16)]
        py = pyv[pl.ds(i * 16, 16)]
        lo = lov[pl.ds(i * 16, 16)]
        hi = hiv[pl.ds(i * 16, 16)]
        rad = radv[pl.ds(i * 16, 16)]

        best_a = big
        best_i = jnp.zeros((16,), jnp.int32)
        for m in range(M):
            bx0 = boxv[pl.ds(0 * ROWLEN + m * 16, 16)]
            by0 = boxv[pl.ds(1 * ROWLEN + m * 16, 16)]
            bx1 = boxv[pl.ds(2 * ROWLEN + m * 16, 16)]
            by1 = boxv[pl.ds(3 * ROWLEN + m * 16, 16)]
            bcx = boxv[pl.ds(4 * ROWLEN + m * 16, 16)]
            bcy = boxv[pl.ds(5 * ROWLEN + m * 16, 16)]
            l = px - bx0
            t = py - by0
            r = bx1 - px
            b = by1 - py
            omin = jnp.minimum(jnp.minimum(l, t), jnp.minimum(r, b))
            omax = jnp.maximum(jnp.maximum(l, t), jnp.maximum(r, b))
            m_in = omin > 0.0
            m_lvl = (omax > lo) & (omax < hi)
            adx = jnp.abs(px - bcx)
            ady = jnp.abs(py - bcy)
            m_ctr = jnp.maximum(adx, ady) < rad
            mask = m_in & m_lvl & m_ctr
            area = (l + r) * (t + b)
            a = jnp.where(mask, area, big)
            upd = a < best_a
            best_a = jnp.where(upd, a, best_a)
            best_i = jnp.where(upd, jnp.full((16,), m, jnp.int32), best_i)

        gidx = best_i * 16 + lane
        x0g = plsc.load_gather(boxv, [gidx])
        y0g = plsc.load_gather(boxv, [gidx + ROWLEN])
        x1g = plsc.load_gather(boxv, [gidx + 2 * ROWLEN])
        y1g = plsc.load_gather(boxv, [gidx + 3 * ROWLEN])
        clg = plsc.load_gather(boxv, [gidx + 6 * ROWLEN])

        mask2 = best_a < big
        l2 = px - x0g
        t2 = py - y0g
        r2 = x1g - px
        b2 = y1g - py
        lrmin = jnp.minimum(l2, r2)
        lrmax = jnp.maximum(l2, r2)
        tbmin = jnp.minimum(t2, b2)
        tbmax = jnp.maximum(t2, b2)
        ratio = lrmin * tbmin / (lrmax * tbmax + 1e-10)
        ratio = jnp.where(mask2, ratio, jnp.ones((16,), jnp.float32))
        yb = jnp.full((16,), 0x5f3759df, jnp.int32) - (
            plsc.bitcast(ratio, jnp.int32) >> 1)
        y = plsc.bitcast(yb, jnp.float32)
        half_x = ratio * 0.5
        for _ in range(3):
            y = y * (1.5 - half_x * y * y)
        cnt = ratio * y

        neg1 = jnp.full((16,), -1.0, jnp.float32)
        clsv[pl.ds(i * 16, 16)] = jnp.where(
            mask2, clg.astype(jnp.int32), jnp.zeros((16,), jnp.int32))
        cntv[pl.ds(i * 16, 16)] = jnp.where(mask2, cnt, neg1)
        lv[pl.ds(i * 16, 16)] = jnp.where(mask2, l2, neg1)
        tv[pl.ds(i * 16, 16)] = jnp.where(mask2, t2, neg1)
        rv[pl.ds(i * 16, 16)] = jnp.where(mask2, r2, neg1)
        bv[pl.ds(i * 16, 16)] = jnp.where(mask2, b2, neg1)
        return 0

    lax.fori_loop(0, NVEC, body, 0)

    pltpu.sync_copy(clsv, cls_o.at[pl.ds(ooff, CHUNK)])
    pltpu.sync_copy(cntv, cnt_o.at[pl.ds(ooff, CHUNK)])
    pltpu.sync_copy(lv, l_o.at[pl.ds(ooff, CHUNK)])
    pltpu.sync_copy(tv, t_o.at[pl.ds(ooff, CHUNK)])
    pltpu.sync_copy(rv, r_o.at[pl.ds(ooff, CHUNK)])
    pltpu.sync_copy(bv, b_o.at[pl.ds(ooff, CHUNK)])


def _build_sc_call():
    mesh = plsc.VectorSubcoreMesh(core_axis_name="c", subcore_axis_name="s")
    f32 = jnp.float32
    out_type = [
        jax.ShapeDtypeStruct((B * NPAD,), jnp.int32),
        jax.ShapeDtypeStruct((B * NPAD,), f32),
        jax.ShapeDtypeStruct((B * NPAD,), f32),
        jax.ShapeDtypeStruct((B * NPAD,), f32),
        jax.ShapeDtypeStruct((B * NPAD,), f32),
        jax.ShapeDtypeStruct((B * NPAD,), f32),
    ]
    scratch_types = [
        pltpu.VMEM((CHUNK,), f32),
        pltpu.VMEM((CHUNK,), f32),
        pltpu.VMEM((CHUNK,), f32),
        pltpu.VMEM((CHUNK,), f32),
        pltpu.VMEM((CHUNK,), f32),
        pltpu.VMEM((BOXSZ,), f32),
        pltpu.VMEM((CHUNK,), jnp.int32),
        pltpu.VMEM((CHUNK,), f32),
        pltpu.VMEM((CHUNK,), f32),
        pltpu.VMEM((CHUNK,), f32),
        pltpu.VMEM((CHUNK,), f32),
        pltpu.VMEM((CHUNK,), f32),
    ]
    return pl.kernel(_tile_body, mesh=mesh, out_type=out_type,
                     scratch_types=scratch_types,
                     compiler_params=pltpu.CompilerParams(
                         needs_layout_passes=False))


_sc_call_cache = []


def _get_sc_call():
    if not _sc_call_cache:
        _sc_call_cache.append(_build_sc_call())
    return _sc_call_cache[0]


@jax.jit
def _run(gt_boxes, classes):
    px, py, lo, hi, rad = _point_data()
    x0 = gt_boxes[..., 0]
    y0 = gt_boxes[..., 1]
    x1 = gt_boxes[..., 2]
    y1 = gt_boxes[..., 3]
    cx = (x0 + x1) / 2
    cy = (y0 + y1) / 2
    rows = jnp.stack([x0, y0, x1, y1, cx, cy, classes.astype(jnp.float32)],
                     axis=1)
    box = jnp.broadcast_to(rows[..., None], (B, NROW, M, 16))
    box = box.reshape(B * BOXSZ)

    cls_f, cnt_f, l_f, t_f, r_f, b_f = _get_sc_call()(px, py, lo, hi, rad, box)

    cls_t = cls_f.reshape(B, NPAD)[:, :TOT, None]
    cnt_t = cnt_f.reshape(B, NPAD)[:, :TOT, None]
    reg_t = jnp.stack([a.reshape(B, NPAD)[:, :TOT]
                       for a in (l_f, t_f, r_f, b_f)], axis=-1)
    return cls_t, cnt_t, reg_t


def kernel(gt_boxes, classes, cls_logits_0, cnt_logits_0, reg_preds_0,
           cls_logits_1, cnt_logits_1, reg_preds_1,
           cls_logits_2, cnt_logits_2, reg_preds_2,
           cls_logits_3, cnt_logits_3, reg_preds_3,
           cls_logits_4, cnt_logits_4, reg_preds_4):
    return _run(gt_boxes, classes)


# speedup vs baseline: 5.8530x; 1.7030x over previous
import functools

import jax
import jax.numpy as jnp
from jax import lax
from jax.experimental import pallas as pl
from jax.experimental.pallas import tpu as pltpu
from jax.experimental.pallas import tpu_sc as plsc

STRIDES = (8, 16, 32, 64, 128)
LIMITS = ((-1.0, 64.0), (64.0, 128.0), (128.0, 256.0), (256.0, 512.0),
          (512.0, 999999.0))
IMG = 1024
B = 8
M = 50
MP = 64
RADIU_RATIO = 1.5
BIG = 999999999.0

LVLN = tuple((IMG // s) ** 2 for s in STRIDES)
TOT = sum(LVLN)
LVLSTART = tuple(sum(LVLN[:i]) for i in range(5))
Q = tuple(n // 4 for n in LVLN)
SEGSTART = tuple(sum(Q[:i]) for i in range(5))
SEGVECS = tuple(q // 16 for q in Q)
CHUNK = TOT // 4
ROWLEN = M * 16
COMPOFF = 7 * ROWLEN
BOXSZ = COMPOFF + 4 * MP


def _point_data():
    xs, ys = [], []
    for s in STRIDES:
        hw = IMG // s
        shifts = jnp.arange(0, hw * s, s, dtype=jnp.float32) + s // 2
        sy, sx = jnp.meshgrid(shifts, shifts, indexing='ij')
        xs.append(sx.reshape(-1))
        ys.append(sy.reshape(-1))
    return jnp.concatenate(xs), jnp.concatenate(ys)


def _tile_body(px_h, py_h, box_h,
               cls_o, cnt_o, l_o, t_o, r_o, b_o,
               pxv, pyv, boxv, listv,
               clsv, cntv, lv, tv, rv, bv):
    wid = lax.axis_index("s") * 2 + lax.axis_index("c")
    bi = wid // 4
    k = wid % 4

    pltpu.sync_copy(box_h.at[pl.ds(bi * BOXSZ, BOXSZ)], boxv)
    for L in range(5):
        src = LVLSTART[L] + k * Q[L]
        pltpu.sync_copy(px_h.at[pl.ds(src, Q[L])],
                        pxv.at[pl.ds(SEGSTART[L], Q[L])])
        pltpu.sync_copy(py_h.at[pl.ds(src, Q[L])],
                        pyv.at[pl.ds(SEGSTART[L], Q[L])])

    lane = lax.broadcasted_iota(jnp.int32, (16,), 0)
    big = jnp.full((16,), BIG, jnp.float32)
    zeros_i = jnp.zeros((16,), jnp.int32)

    mw, gm = [], []
    for g in range(4):
        x0c = boxv[pl.ds(COMPOFF + 0 * MP + g * 16, 16)]
        y0c = boxv[pl.ds(COMPOFF + 1 * MP + g * 16, 16)]
        x1c = boxv[pl.ds(COMPOFF + 2 * MP + g * 16, 16)]
        y1c = boxv[pl.ds(COMPOFF + 3 * MP + g * 16, 16)]
        mw.append(jnp.maximum(x1c - x0c, y1c - y0c))
        gm.append(lane + g * 16)
    cnts = []
    for L in range(5):
        lo, hi = LIMITS[L]
        ccv = zeros_i
        for g in range(4):
            valid = ((mw[g] > lo - 1.0) & (mw[g] * 0.5 < hi + 1.0)
                     & (gm[g] < M))
            pos = plsc.cumsum(valid.astype(jnp.int32))
            idx = jnp.full((16,), L * MP, jnp.int32) + ccv + pos - 1
            plsc.store_scatter(listv, [idx], gm[g], mask=valid)
            ccv = ccv + plsc.all_reduce_population_count(valid)
        cnts.append(jnp.max(ccv))

    for L in range(5):
        lov = jnp.full((16,), LIMITS[L][0], jnp.float32)
        hiv = jnp.full((16,), LIMITS[L][1], jnp.float32)
        radv = jnp.full((16,), STRIDES[L] * RADIU_RATIO, jnp.float32)
        base = SEGSTART[L]
        n_l = cnts[L]

        def seg_body(i, _, base=base, lov=lov, hiv=hiv, radv=radv, L=L,
                     n_l=n_l):
            px = pxv[pl.ds(base + i * 16, 16)]
            py = pyv[pl.ds(base + i * 16, 16)]

            def box_body(j, carry):
                best_a, best_i = carry
                jv = jnp.full((16,), L * MP, jnp.int32) + j
                m_splat = plsc.load_gather(listv, [jv])
                gi = m_splat * 16 + lane
                bx0 = plsc.load_gather(boxv, [gi])
                by0 = plsc.load_gather(boxv, [gi + 800])
                bx1 = plsc.load_gather(boxv, [gi + 1600])
                by1 = plsc.load_gather(boxv, [gi + 2400])
                bcx = plsc.load_gather(boxv, [gi + 3200])
                bcy = plsc.load_gather(boxv, [gi + 4000])
                l = px - bx0
                t = py - by0
                r = bx1 - px
                b = by1 - py
                omin = jnp.minimum(jnp.minimum(l, t), jnp.minimum(r, b))
                omax = jnp.maximum(jnp.maximum(l, t), jnp.maximum(r, b))
                m_in = omin > 0.0
                m_lvl = (omax > lov) & (omax < hiv)
                adx = jnp.abs(px - bcx)
                ady = jnp.abs(py - bcy)
                m_ctr = jnp.maximum(adx, ady) < radv
                mask = m_in & m_lvl & m_ctr
                area = (l + r) * (t + b)
                a = jnp.where(mask, area, big)
                upd = a < best_a
                return (jnp.where(upd, a, best_a),
                        jnp.where(upd, m_splat, best_i))

            best_a, best_i = lax.fori_loop(0, n_l, box_body,
                                           (big, zeros_i))

            gidx = best_i * 16 + lane
            x0g = plsc.load_gather(boxv, [gidx])
            y0g = plsc.load_gather(boxv, [gidx + 800])
            x1g = plsc.load_gather(boxv, [gidx + 1600])
            y1g = plsc.load_gather(boxv, [gidx + 2400])
            clg = plsc.load_gather(boxv, [gidx + 4800])

            mask2 = best_a < big
            l2 = px - x0g
            t2 = py - y0g
            r2 = x1g - px
            b2 = y1g - py
            lrmin = jnp.minimum(l2, r2)
            lrmax = jnp.maximum(l2, r2)
            tbmin = jnp.minimum(t2, b2)
            tbmax = jnp.maximum(t2, b2)
            ratio = lrmin * tbmin / (lrmax * tbmax + 1e-10)
            ratio = jnp.where(mask2, ratio, jnp.ones((16,), jnp.float32))
            yb = jnp.full((16,), 0x5f3759df, jnp.int32) - (
                plsc.bitcast(ratio, jnp.int32) >> 1)
            y = plsc.bitcast(yb, jnp.float32)
            half_x = ratio * 0.5
            for _ in range(3):
                y = y * (1.5 - half_x * y * y)
            cnt = ratio * y

            neg1 = jnp.full((16,), -1.0, jnp.float32)
            sl = pl.ds(base + i * 16, 16)
            clsv[sl] = jnp.where(mask2, clg.astype(jnp.int32), zeros_i)
            cntv[sl] = jnp.where(mask2, cnt, neg1)
            lv[sl] = jnp.where(mask2, l2, neg1)
            tv[sl] = jnp.where(mask2, t2, neg1)
            rv[sl] = jnp.where(mask2, r2, neg1)
            bv[sl] = jnp.where(mask2, b2, neg1)
            return 0

        lax.fori_loop(0, SEGVECS[L], seg_body, 0)

    for L in range(5):
        dst = bi * TOT + LVLSTART[L] + k * Q[L]
        s_in = pl.ds(SEGSTART[L], Q[L])
        s_out = pl.ds(dst, Q[L])
        pltpu.sync_copy(clsv.at[s_in], cls_o.at[s_out])
        pltpu.sync_copy(cntv.at[s_in], cnt_o.at[s_out])
        pltpu.sync_copy(lv.at[s_in], l_o.at[s_out])
        pltpu.sync_copy(tv.at[s_in], t_o.at[s_out])
        pltpu.sync_copy(rv.at[s_in], r_o.at[s_out])
        pltpu.sync_copy(bv.at[s_in], b_o.at[s_out])


def _build_sc_call():
    mesh = plsc.VectorSubcoreMesh(core_axis_name="c", subcore_axis_name="s")
    f32 = jnp.float32
    out_type = [
        jax.ShapeDtypeStruct((B * TOT,), jnp.int32),
        jax.ShapeDtypeStruct((B * TOT,), f32),
        jax.ShapeDtypeStruct((B * TOT,), f32),
        jax.ShapeDtypeStruct((B * TOT,), f32),
        jax.ShapeDtypeStruct((B * TOT,), f32),
        jax.ShapeDtypeStruct((B * TOT,), f32),
    ]
    scratch_types = [
        pltpu.VMEM((CHUNK,), f32),
        pltpu.VMEM((CHUNK,), f32),
        pltpu.VMEM((BOXSZ,), f32),
        pltpu.VMEM((5 * MP + 16,), jnp.int32),
        pltpu.VMEM((CHUNK,), jnp.int32),
        pltpu.VMEM((CHUNK,), f32),
        pltpu.VMEM((CHUNK,), f32),
        pltpu.VMEM((CHUNK,), f32),
        pltpu.VMEM((CHUNK,), f32),
        pltpu.VMEM((CHUNK,), f32),
    ]
    return pl.kernel(_tile_body, mesh=mesh, out_type=out_type,
                     scratch_types=scratch_types,
                     compiler_params=pltpu.CompilerParams(
                         needs_layout_passes=False))


_sc_call_cache = []


def _get_sc_call():
    if not _sc_call_cache:
        _sc_call_cache.append(_build_sc_call())
    return _sc_call_cache[0]


@jax.jit
def _run(gt_boxes, classes):
    px, py = _point_data()
    x0 = gt_boxes[..., 0]
    y0 = gt_boxes[..., 1]
    x1 = gt_boxes[..., 2]
    y1 = gt_boxes[..., 3]
    cx = (x0 + x1) / 2
    cy = (y0 + y1) / 2
    rows = jnp.stack([x0, y0, x1, y1, cx, cy, classes.astype(jnp.float32)],
                     axis=1)
    rep = jnp.broadcast_to(rows[..., None], (B, 7, M, 16)).reshape(B, -1)
    comp = jnp.pad(jnp.stack([x0, y0, x1, y1], axis=1),
                   ((0, 0), (0, 0), (0, MP - M))).reshape(B, -1)
    box = jnp.concatenate([rep, comp], axis=-1).reshape(-1)

    cls_f, cnt_f, l_f, t_f, r_f, b_f = _get_sc_call()(px, py, box)

    cls_t = cls_f.reshape(B, TOT)[:, :, None]
    cnt_t = cnt_f.reshape(B, TOT)[:, :, None]
    reg_t = jnp.stack([a.reshape(B, TOT) for a in (l_f, t_f, r_f, b_f)],
                      axis=-1)
    return cls_t, cnt_t, reg_t


def kernel(gt_boxes, classes, cls_logits_0, cnt_logits_0, reg_preds_0,
           cls_logits_1, cnt_logits_1, reg_preds_1,
           cls_logits_2, cnt_logits_2, reg_preds_2,
           cls_logits_3, cnt_logits_3, reg_preds_3,
           cls_logits_4, cnt_logits_4, reg_preds_4):
    return _run(gt_boxes, classes)
